# Initial kernel scaffold; baseline (speedup 1.0000x reference)
import jax
import jax.numpy as jnp
from jax.experimental import pallas as pl


def kernel(features):
    raise NotImplementedError("write your pallas kernel here")


# speedup vs baseline: 17.0215x; 17.0215x over previous
import functools

import jax
import jax.numpy as jnp
import numpy as np
from jax import lax
from jax.experimental import pallas as pl
from jax.experimental.pallas import tpu as pltpu
from jax.experimental.pallas import tpu_sc as plsc

_K = 16384
_N = 512 * 6 * 4096
_NWORK = 32
_NBKT = 4096
_CAP = 4096
_MIN32 = np.int32(-2147483648)
_POS = np.int32(2147483647)


def _umap(xi):
    return jnp.where(xi < 0, ~xi, xi | _MIN32)


def _k1_body(x_hbm, hist_hbm, buf, hist1, outv):
    n = x_hbm.shape[0]
    shard = n // _NWORK
    win = 16384
    nwin = shard // win
    nvec_u = win // (16 * 8)

    wid = lax.axis_index("s") * 2 + lax.axis_index("c")
    base = wid * shard

    zeros16 = jnp.zeros((16,), jnp.int32)
    ones16 = jnp.ones((16,), jnp.int32)
    laneoff = lax.iota(jnp.int32, 16) * _NBKT

    def zero_body(i, _):
        for r in range(16):
            hist1[pl.ds((i * 16 + r) * 16, 16)] = zeros16
        return 0

    lax.fori_loop(0, (16 * _NBKT) // 256, zero_body, 0)

    def win_body(w, _):
        pltpu.sync_copy(x_hbm.at[pl.ds(base + w * win, win)], buf)

        def vec_body(i, _):
            for u in range(8):
                xi = buf[pl.ds((i * 8 + u) * 16, 16)]
                m = _umap(xi)
                bkt = lax.shift_right_logical(m, 20)
                plsc.addupdate_scatter(hist1, [laneoff + bkt], ones16)
            return 0

        lax.fori_loop(0, nvec_u, vec_body, 0)
        return 0

    lax.fori_loop(0, nwin, win_body, 0)

    def red_body(g, _):
        acc = hist1[pl.ds(g * 16, 16)]
        for r in range(1, 16):
            acc = acc + hist1[pl.ds(r * _NBKT + g * 16, 16)]
        outv[pl.ds(g * 16, 16)] = acc
        return 0

    lax.fori_loop(0, _NBKT // 16, red_body, 0)
    pltpu.sync_copy(outv, hist_hbm.at[wid])


def _k2_body(hist_ref, b_ref, a_ref):
    merged = jnp.sum(hist_ref[...], axis=0, keepdims=True)
    bidx = lax.broadcasted_iota(jnp.int32, (1, _NBKT), 1)

    def above(b):
        return jnp.sum(jnp.where(bidx > b, merged, 0))

    def bis(_, carry):
        lo, hi = carry
        mid = (lo + hi) // 2
        less = above(mid) < _K
        return (jnp.where(less, lo, mid), jnp.where(less, mid, hi))

    lo0 = jnp.int32(-1)
    hi0 = jnp.int32(_NBKT - 1)
    _, bstar = lax.fori_loop(0, 12, bis, (lo0, hi0))
    a = above(bstar)
    b_ref[...] = jnp.full((128,), bstar, jnp.int32)
    a_ref[...] = jnp.full((128,), a, jnp.int32)


def _k3_body(x_hbm, b_hbm, cand_hbm, cidx_hbm, cnt_hbm,
             buf, cand, cidx, bvec, cntv):
    n = x_hbm.shape[0]
    shard = n // _NWORK
    win = 32768
    nwin = shard // win
    nvec_u = win // (16 * 4)

    wid = lax.axis_index("s") * 2 + lax.axis_index("c")
    base = wid * shard

    pltpu.sync_copy(b_hbm.at[pl.ds(0, 16)], bvec)
    bv = bvec[...]
    lanes = lax.iota(jnp.int32, 16)

    def win_body(w, off):
        pltpu.sync_copy(x_hbm.at[pl.ds(base + w * win, win)], buf)
        wbase = base + w * win

        def vec_body(i, off):
            for u in range(4):
                xi = buf[pl.ds((i * 4 + u) * 16, 16)]
                m = _umap(xi)
                bkt = lax.shift_right_logical(m, 20)
                sel = bkt == bv
                offc = jnp.minimum(off, _CAP - 16)
                plsc.store_compressed(cand.at[pl.ds(offc, 16)], m, mask=sel)
                fidx = (wbase + (i * 4 + u) * 16) + lanes
                plsc.store_compressed(cidx.at[pl.ds(offc, 16)], fidx, mask=sel)
                off = off + plsc.all_reduce_population_count(sel)[0]
            return off

        return lax.fori_loop(0, nvec_u, vec_body, off)

    off = lax.fori_loop(0, nwin, win_body, jnp.int32(0))
    cnt = jnp.minimum(off, _CAP)

    def cnt_body(t, _):
        cntv[pl.ds(t * 16, 16)] = jnp.full((16,), cnt, jnp.int32)
        return 0

    lax.fori_loop(0, 8, cnt_body, 0)
    pltpu.sync_copy(cand, cand_hbm.at[wid])
    pltpu.sync_copy(cidx, cidx_hbm.at[wid])
    pltpu.sync_copy(cntv, cnt_hbm.at[wid])


def _k4_body(b_smem, a_smem, cand_ref, cidx_ref, cnt_ref, t_ref, c_ref):
    bstar = b_smem[0]
    above = a_smem[0]
    j = _K - above

    cnt0 = cnt_ref[...][:, 0:1]
    valid = lax.broadcasted_iota(jnp.int32, (_NWORK, _CAP), 1) < cnt0
    low = jnp.where(valid, cand_ref[...] & 0xFFFFF, -1)

    def count_ge(t):
        return jnp.sum(jnp.where(low >= t, 1, 0).astype(jnp.int32))

    def bis(_, carry):
        lo, hi = carry
        mid = (lo + hi) // 2
        ge = count_ge(mid) >= j
        return (jnp.where(ge, mid, lo), jnp.where(ge, hi, mid))

    tlow, _ = lax.fori_loop(0, 20, bis, (jnp.int32(0), jnp.int32(1 << 20)))

    greater = above + count_ge(tlow + 1)
    t_extra = _K - greater
    eqidx = jnp.where(valid & (low == tlow), cidx_ref[...], _POS)

    def count_le(c):
        return jnp.sum(jnp.where(eqidx <= c, 1, 0).astype(jnp.int32))

    def bis_idx(_, carry):
        lo, hi = carry
        mid = (lo + hi) // 2
        ge = count_le(mid) >= t_extra
        return (jnp.where(ge, lo, mid), jnp.where(ge, mid, hi))

    _, cutoff = lax.fori_loop(
        0, 24, bis_idx, (jnp.int32(-1), jnp.int32(_N - 1)))

    u_t = (bstar << 20) | tlow
    t_ref[...] = jnp.full((128,), u_t ^ _MIN32, jnp.int32)
    c_ref[...] = jnp.full((128,), cutoff, jnp.int32)


def _k5_body(t_smem, c_smem, x_ref, o_ref):
    ts = t_smem[0]
    cutoff = c_smem[0]
    x = x_ref[...]
    rows, d = x_ref.shape
    xi = pltpu.bitcast(x, jnp.int32)
    us = jnp.where(xi < 0, xi ^ _POS, xi)
    base = pl.program_id(0) * (rows * d)
    fidx = (base
            + lax.broadcasted_iota(jnp.int32, (rows, d), 0) * d
            + lax.broadcasted_iota(jnp.int32, (rows, d), 1))
    keep = (us > ts) | ((us == ts) & (fidx <= cutoff))
    o_ref[...] = jnp.where(keep, x, jnp.float32(0.0))


def kernel(features):
    b, l, d = features.shape
    n = b * l * d
    flat = features.reshape(n)
    flat_i = lax.bitcast_convert_type(flat, jnp.int32)
    mesh = plsc.VectorSubcoreMesh(
        core_axis_name="c", subcore_axis_name="s", num_cores=2, num_subcores=16
    )

    k1 = functools.partial(
        pl.kernel,
        out_type=jax.ShapeDtypeStruct((_NWORK, _NBKT), jnp.int32),
        mesh=mesh,
        scratch_types=[
            pltpu.VMEM((16384,), jnp.int32),
            pltpu.VMEM((16 * _NBKT,), jnp.int32),
            pltpu.VMEM((_NBKT,), jnp.int32),
        ],
        compiler_params=pltpu.CompilerParams(needs_layout_passes=False),
    )(_k1_body)
    hist = k1(flat_i)

    b_rep, a_rep = pl.pallas_call(
        _k2_body,
        out_shape=(
            jax.ShapeDtypeStruct((128,), jnp.int32),
            jax.ShapeDtypeStruct((128,), jnp.int32),
        ),
    )(hist)

    k3 = functools.partial(
        pl.kernel,
        out_type=(
            jax.ShapeDtypeStruct((_NWORK, _CAP), jnp.int32),
            jax.ShapeDtypeStruct((_NWORK, _CAP), jnp.int32),
            jax.ShapeDtypeStruct((_NWORK, 128), jnp.int32),
        ),
        mesh=mesh,
        scratch_types=[
            pltpu.VMEM((32768,), jnp.int32),
            pltpu.VMEM((_CAP,), jnp.int32),
            pltpu.VMEM((_CAP,), jnp.int32),
            pltpu.VMEM((16,), jnp.int32),
            pltpu.VMEM((128,), jnp.int32),
        ],
        compiler_params=pltpu.CompilerParams(needs_layout_passes=False),
    )(_k3_body)
    cand, cidx, cnt = k3(flat_i, b_rep)

    tvec, cvec = pl.pallas_call(
        _k4_body,
        in_specs=[
            pl.BlockSpec(memory_space=pltpu.SMEM),
            pl.BlockSpec(memory_space=pltpu.SMEM),
            pl.BlockSpec(memory_space=pltpu.VMEM),
            pl.BlockSpec(memory_space=pltpu.VMEM),
            pl.BlockSpec(memory_space=pltpu.VMEM),
        ],
        out_shape=(
            jax.ShapeDtypeStruct((128,), jnp.int32),
            jax.ShapeDtypeStruct((128,), jnp.int32),
        ),
    )(b_rep, a_rep, cand, cidx, cnt)

    rows = b * l
    blk = 8
    out = pl.pallas_call(
        _k5_body,
        grid=(rows // blk,),
        in_specs=[
            pl.BlockSpec(memory_space=pltpu.SMEM),
            pl.BlockSpec(memory_space=pltpu.SMEM),
            pl.BlockSpec((blk, d), lambda i: (i, 0)),
        ],
        out_specs=pl.BlockSpec((blk, d), lambda i: (i, 0)),
        out_shape=jax.ShapeDtypeStruct((rows, d), jnp.float32),
    )(tvec, cvec, flat.reshape(rows, d))
    return out.reshape(b, l, d)


# speedup vs baseline: 21.0591x; 1.2372x over previous
import functools

import jax
import jax.numpy as jnp
import numpy as np
from jax import lax
from jax.experimental import pallas as pl
from jax.experimental.pallas import tpu as pltpu
from jax.experimental.pallas import tpu_sc as plsc

_K = 16384
_N = 512 * 6 * 4096
_NWORK = 32
_NBKT = 4096
_CAP = 4096
_MIN32 = np.int32(-2147483648)
_POS = np.int32(2147483647)


def _umap(xi):
    return jnp.where(xi < 0, ~xi, xi | _MIN32)


def _k1_body(x_hbm, hist_hbm, buf, buf2, hist1, outv, sem0, sem1):
    n = x_hbm.shape[0]
    shard = n // _NWORK
    win = 16384
    nwin = shard // win
    nvec_u = win // (16 * 8)

    wid = lax.axis_index("s") * 2 + lax.axis_index("c")
    base = wid * shard

    zeros16 = jnp.zeros((16,), jnp.int32)
    ones16 = jnp.ones((16,), jnp.int32)
    laneoff = lax.iota(jnp.int32, 16) * _NBKT

    def zero_body(i, _):
        for r in range(16):
            hist1[pl.ds((i * 16 + r) * 16, 16)] = zeros16
        return 0

    lax.fori_loop(0, (16 * _NBKT) // 256, zero_body, 0)

    bufs = (buf, buf2)
    sems = (sem0, sem1)

    def start(w, b):
        pltpu.async_copy(x_hbm.at[pl.ds(base + w * win, win)], bufs[b], sems[b])

    def wait(w, b):
        pltpu.make_async_copy(
            x_hbm.at[pl.ds(base + w * win, win)], bufs[b], sems[b]).wait()

    start(0, 0)

    def win2_body(w2, _):
        for b in range(2):
            w = w2 * 2 + b

            @pl.when(w + 1 < nwin)
            def _():
                start(w + 1, (b + 1) % 2)

            wait(w, b)
            cur = bufs[b]

            def vec_body(i, _):
                for u in range(8):
                    xi = cur[pl.ds((i * 8 + u) * 16, 16)]
                    m = _umap(xi)
                    bkt = lax.shift_right_logical(m, 20)
                    plsc.addupdate_scatter(hist1, [laneoff + bkt], ones16)
                return 0

            lax.fori_loop(0, nvec_u, vec_body, 0)
        return 0

    lax.fori_loop(0, nwin // 2, win2_body, 0)

    def red_body(g, _):
        acc = hist1[pl.ds(g * 16, 16)]
        for r in range(1, 16):
            acc = acc + hist1[pl.ds(r * _NBKT + g * 16, 16)]
        outv[pl.ds(g * 16, 16)] = acc
        return 0

    lax.fori_loop(0, _NBKT // 16, red_body, 0)
    pltpu.sync_copy(outv, hist_hbm.at[wid])


def _k2_body(hist_ref, b_ref, a_ref):
    merged = jnp.sum(hist_ref[...], axis=0, keepdims=True)
    bidx = lax.broadcasted_iota(jnp.int32, (1, _NBKT), 1)

    def above(b):
        return jnp.sum(jnp.where(bidx > b, merged, 0))

    def bis(_, carry):
        lo, hi = carry
        mid = (lo + hi) // 2
        less = above(mid) < _K
        return (jnp.where(less, lo, mid), jnp.where(less, mid, hi))

    lo0 = jnp.int32(-1)
    hi0 = jnp.int32(_NBKT - 1)
    _, bstar = lax.fori_loop(0, 12, bis, (lo0, hi0))
    a = above(bstar)
    b_ref[...] = jnp.full((128,), bstar, jnp.int32)
    a_ref[...] = jnp.full((128,), a, jnp.int32)


def _k3_body(x_hbm, b_hbm, cand_hbm, cidx_hbm, cnt_hbm,
             buf, buf2, cand, cidx, bvec, cntv, sem0, sem1):
    n = x_hbm.shape[0]
    shard = n // _NWORK
    win = 32768
    nwin = shard // win
    nvec_u = win // (16 * 4)

    wid = lax.axis_index("s") * 2 + lax.axis_index("c")
    base = wid * shard

    pltpu.sync_copy(b_hbm.at[pl.ds(0, 16)], bvec)
    bv = bvec[...]
    lanes = lax.iota(jnp.int32, 16)

    bufs = (buf, buf2)
    sems = (sem0, sem1)

    def start(w, b):
        pltpu.async_copy(x_hbm.at[pl.ds(base + w * win, win)], bufs[b], sems[b])

    def wait(w, b):
        pltpu.make_async_copy(
            x_hbm.at[pl.ds(base + w * win, win)], bufs[b], sems[b]).wait()

    start(0, 0)

    def win2_body(w2, off):
        for b in range(2):
            w = w2 * 2 + b

            @pl.when(w + 1 < nwin)
            def _():
                start(w + 1, (b + 1) % 2)

            wait(w, b)
            cur = bufs[b]
            wbase = base + w * win

            def vec_body(i, off):
                for u in range(4):
                    xi = cur[pl.ds((i * 4 + u) * 16, 16)]
                    m = _umap(xi)
                    bkt = lax.shift_right_logical(m, 20)
                    sel = bkt == bv
                    offc = jnp.minimum(off, _CAP - 16)
                    plsc.store_compressed(cand.at[pl.ds(offc, 16)], m, mask=sel)
                    fidx = (wbase + (i * 4 + u) * 16) + lanes
                    plsc.store_compressed(cidx.at[pl.ds(offc, 16)], fidx,
                                          mask=sel)
                    off = off + plsc.all_reduce_population_count(sel)[0]
                return off

            off = lax.fori_loop(0, nvec_u, vec_body, off)
        return off

    off = lax.fori_loop(0, nwin // 2, win2_body, jnp.int32(0))
    cnt = jnp.minimum(off, _CAP)

    def cnt_body(t, _):
        cntv[pl.ds(t * 16, 16)] = jnp.full((16,), cnt, jnp.int32)
        return 0

    lax.fori_loop(0, 8, cnt_body, 0)
    pltpu.sync_copy(cand, cand_hbm.at[wid])
    pltpu.sync_copy(cidx, cidx_hbm.at[wid])
    pltpu.sync_copy(cntv, cnt_hbm.at[wid])


def _k4_body(b_smem, a_smem, cand_ref, cidx_ref, cnt_ref, t_ref, c_ref):
    bstar = b_smem[0]
    above = a_smem[0]
    j = _K - above

    cnt0 = cnt_ref[...][:, 0:1]
    valid = lax.broadcasted_iota(jnp.int32, (_NWORK, _CAP), 1) < cnt0
    low = jnp.where(valid, cand_ref[...] & 0xFFFFF, -1)

    def count_ge(t):
        return jnp.sum(jnp.where(low >= t, 1, 0).astype(jnp.int32))

    def bis(_, carry):
        lo, hi = carry
        mid = (lo + hi) // 2
        ge = count_ge(mid) >= j
        return (jnp.where(ge, mid, lo), jnp.where(ge, hi, mid))

    tlow, _ = lax.fori_loop(0, 20, bis, (jnp.int32(0), jnp.int32(1 << 20)))

    greater = above + count_ge(tlow + 1)
    t_extra = _K - greater
    eqidx = jnp.where(valid & (low == tlow), cidx_ref[...], _POS)

    def count_le(c):
        return jnp.sum(jnp.where(eqidx <= c, 1, 0).astype(jnp.int32))

    def bis_idx(_, carry):
        lo, hi = carry
        mid = (lo + hi) // 2
        ge = count_le(mid) >= t_extra
        return (jnp.where(ge, lo, mid), jnp.where(ge, mid, hi))

    _, cutoff = lax.fori_loop(
        0, 24, bis_idx, (jnp.int32(-1), jnp.int32(_N - 1)))

    u_t = (bstar << 20) | tlow
    t_ref[...] = jnp.full((128,), u_t ^ _MIN32, jnp.int32)
    c_ref[...] = jnp.full((128,), cutoff, jnp.int32)


def _k5_body(t_smem, c_smem, x_ref, o_ref):
    ts = t_smem[0]
    cutoff = c_smem[0]
    x = x_ref[...]
    rows, d = x_ref.shape
    xi = pltpu.bitcast(x, jnp.int32)
    us = jnp.where(xi < 0, xi ^ _POS, xi)
    base = pl.program_id(0) * (rows * d)
    fidx = (base
            + lax.broadcasted_iota(jnp.int32, (rows, d), 0) * d
            + lax.broadcasted_iota(jnp.int32, (rows, d), 1))
    keep = (us > ts) | ((us == ts) & (fidx <= cutoff))
    o_ref[...] = jnp.where(keep, x, jnp.float32(0.0))


def kernel(features):
    b, l, d = features.shape
    n = b * l * d
    flat = features.reshape(n)
    flat_i = lax.bitcast_convert_type(flat, jnp.int32)
    mesh = plsc.VectorSubcoreMesh(
        core_axis_name="c", subcore_axis_name="s", num_cores=2, num_subcores=16
    )

    k1 = functools.partial(
        pl.kernel,
        out_type=jax.ShapeDtypeStruct((_NWORK, _NBKT), jnp.int32),
        mesh=mesh,
        scratch_types=[
            pltpu.VMEM((16384,), jnp.int32),
            pltpu.VMEM((16384,), jnp.int32),
            pltpu.VMEM((16 * _NBKT,), jnp.int32),
            pltpu.VMEM((_NBKT,), jnp.int32),
            pltpu.SemaphoreType.DMA,
            pltpu.SemaphoreType.DMA,
        ],
        compiler_params=pltpu.CompilerParams(needs_layout_passes=False),
    )(_k1_body)
    hist = k1(flat_i)

    b_rep, a_rep = pl.pallas_call(
        _k2_body,
        out_shape=(
            jax.ShapeDtypeStruct((128,), jnp.int32),
            jax.ShapeDtypeStruct((128,), jnp.int32),
        ),
    )(hist)

    k3 = functools.partial(
        pl.kernel,
        out_type=(
            jax.ShapeDtypeStruct((_NWORK, _CAP), jnp.int32),
            jax.ShapeDtypeStruct((_NWORK, _CAP), jnp.int32),
            jax.ShapeDtypeStruct((_NWORK, 128), jnp.int32),
        ),
        mesh=mesh,
        scratch_types=[
            pltpu.VMEM((32768,), jnp.int32),
            pltpu.VMEM((_CAP,), jnp.int32),
            pltpu.VMEM((_CAP,), jnp.int32),
            pltpu.VMEM((16,), jnp.int32),
            pltpu.VMEM((128,), jnp.int32),
        ],
        compiler_params=pltpu.CompilerParams(needs_layout_passes=False),
    )(_k3_body)
    cand, cidx, cnt = k3(flat_i, b_rep)

    tvec, cvec = pl.pallas_call(
        _k4_body,
        in_specs=[
            pl.BlockSpec(memory_space=pltpu.SMEM),
            pl.BlockSpec(memory_space=pltpu.SMEM),
            pl.BlockSpec(memory_space=pltpu.VMEM),
            pl.BlockSpec(memory_space=pltpu.VMEM),
            pl.BlockSpec(memory_space=pltpu.VMEM),
        ],
        out_shape=(
            jax.ShapeDtypeStruct((128,), jnp.int32),
            jax.ShapeDtypeStruct((128,), jnp.int32),
        ),
    )(b_rep, a_rep, cand, cidx, cnt)

    rows = b * l
    blk = 64
    out = pl.pallas_call(
        _k5_body,
        grid=(rows // blk,),
        in_specs=[
            pl.BlockSpec(memory_space=pltpu.SMEM),
            pl.BlockSpec(memory_space=pltpu.SMEM),
            pl.BlockSpec((blk, d), lambda i: (i, 0)),
        ],
        out_specs=pl.BlockSpec((blk, d), lambda i: (i, 0)),
        out_shape=jax.ShapeDtypeStruct((rows, d), jnp.float32),
    )(tvec, cvec, flat.reshape(rows, d))
    return out.reshape(b, l, d)


# speedup vs baseline: 40.2159x; 1.9097x over previous
import functools

import jax
import jax.numpy as jnp
import numpy as np
from jax import lax
from jax.experimental import pallas as pl
from jax.experimental.pallas import tpu as pltpu
from jax.experimental.pallas import tpu_sc as plsc

_K = 16384
_N = 512 * 6 * 4096
_NWORK = 32
_NBKT = 4096
_CAP = 4096
_MIN32 = np.int32(-2147483648)
_POS = np.int32(2147483647)


def _umap(xi):
    return jnp.where(xi < 0, ~xi, xi | _MIN32)


def _k1_body(x_hbm, hist_hbm, buf, buf2, hist1, outv, sem0, sem1):
    n = x_hbm.shape[0]
    shard = n // _NWORK
    win = 16384
    nwin = shard // win
    nvec_u = win // (16 * 8)

    wid = lax.axis_index("s") * 2 + lax.axis_index("c")
    base = wid * shard

    zeros16 = jnp.zeros((16,), jnp.int32)
    ones16 = jnp.ones((16,), jnp.int32)
    laneoff = lax.iota(jnp.int32, 16) * _NBKT

    @plsc.parallel_loop(0, (16 * _NBKT) // 16, unroll=8)
    def _(i):
        hist1[pl.ds(i * 16, 16)] = zeros16

    bufs = (buf, buf2)
    sems = (sem0, sem1)

    def start(w, b):
        pltpu.async_copy(x_hbm.at[pl.ds(base + w * win, win)], bufs[b], sems[b])

    def wait(w, b):
        pltpu.make_async_copy(
            x_hbm.at[pl.ds(base + w * win, win)], bufs[b], sems[b]).wait()

    start(0, 0)

    def win2_body(w2, _):
        for b in range(2):
            w = w2 * 2 + b

            @pl.when(w + 1 < nwin)
            def _():
                start(w + 1, (b + 1) % 2)

            wait(w, b)
            cur = bufs[b]

            @plsc.parallel_loop(0, win // 16, unroll=8)
            def _(i):
                xi = cur[pl.ds(i * 16, 16)]
                m = _umap(xi)
                bkt = lax.shift_right_logical(m, 20)
                plsc.addupdate_scatter(hist1, [laneoff + bkt], ones16)
        return 0

    lax.fori_loop(0, nwin // 2, win2_body, 0)

    @plsc.parallel_loop(0, _NBKT // 16, unroll=2)
    def _(g):
        acc = hist1[pl.ds(g * 16, 16)]
        for r in range(1, 16):
            acc = acc + hist1[pl.ds(r * _NBKT + g * 16, 16)]
        outv[pl.ds(g * 16, 16)] = acc
    pltpu.sync_copy(outv, hist_hbm.at[wid])


def _k2_body(hist_ref, b_ref, a_ref):
    merged = jnp.sum(hist_ref[...], axis=0, keepdims=True)
    bidx = lax.broadcasted_iota(jnp.int32, (1, _NBKT), 1)

    def above(b):
        return jnp.sum(jnp.where(bidx > b, merged, 0))

    def bis(_, carry):
        lo, hi = carry
        mid = (lo + hi) // 2
        less = above(mid) < _K
        return (jnp.where(less, lo, mid), jnp.where(less, mid, hi))

    lo0 = jnp.int32(-1)
    hi0 = jnp.int32(_NBKT - 1)
    _, bstar = lax.fori_loop(0, 12, bis, (lo0, hi0))
    a = above(bstar)
    b_ref[...] = jnp.full((128,), bstar, jnp.int32)
    a_ref[...] = jnp.full((128,), a, jnp.int32)


def _k3_body(x_hbm, b_hbm, cand_hbm, cidx_hbm, cnt_hbm,
             buf, buf2, cand, cidx, bvec, cntv, sem0, sem1):
    n = x_hbm.shape[0]
    shard = n // _NWORK
    win = 32768
    nwin = shard // win
    nvec_u = win // (16 * 4)

    wid = lax.axis_index("s") * 2 + lax.axis_index("c")
    base = wid * shard

    pltpu.sync_copy(b_hbm.at[pl.ds(0, 16)], bvec)
    bv = bvec[...]
    lanes = lax.iota(jnp.int32, 16)

    bufs = (buf, buf2)
    sems = (sem0, sem1)

    def start(w, b):
        pltpu.async_copy(x_hbm.at[pl.ds(base + w * win, win)], bufs[b], sems[b])

    def wait(w, b):
        pltpu.make_async_copy(
            x_hbm.at[pl.ds(base + w * win, win)], bufs[b], sems[b]).wait()

    start(0, 0)

    def win2_body(w2, off):
        for b in range(2):
            w = w2 * 2 + b

            @pl.when(w + 1 < nwin)
            def _():
                start(w + 1, (b + 1) % 2)

            wait(w, b)
            cur = bufs[b]
            wbase = base + w * win

            @plsc.parallel_loop(0, win // 16, unroll=4, carry=off)
            def inner(i, off):
                xi = cur[pl.ds(i * 16, 16)]
                m = _umap(xi)
                bkt = lax.shift_right_logical(m, 20)
                sel = bkt == bv
                offc = jnp.minimum(off, _CAP - 16)
                plsc.store_compressed(cand.at[pl.ds(offc, 16)], m, mask=sel)
                fidx = (wbase + i * 16) + lanes
                plsc.store_compressed(cidx.at[pl.ds(offc, 16)], fidx, mask=sel)
                return off + plsc.all_reduce_population_count(sel)[0]

            off = inner
        return off

    off = lax.fori_loop(0, nwin // 2, win2_body, jnp.int32(0))
    cnt = jnp.minimum(off, _CAP)

    def cnt_body(t, _):
        cntv[pl.ds(t * 16, 16)] = jnp.full((16,), cnt, jnp.int32)
        return 0

    lax.fori_loop(0, 8, cnt_body, 0)
    pltpu.sync_copy(cand, cand_hbm.at[wid])
    pltpu.sync_copy(cidx, cidx_hbm.at[wid])
    pltpu.sync_copy(cntv, cnt_hbm.at[wid])


def _k4_body(b_smem, a_smem, cand_ref, cidx_ref, cnt_ref, t_ref, c_ref):
    bstar = b_smem[0]
    above = a_smem[0]
    j = _K - above

    cnt0 = cnt_ref[...][:, 0:1]
    valid = lax.broadcasted_iota(jnp.int32, (_NWORK, _CAP), 1) < cnt0
    low = jnp.where(valid, cand_ref[...] & 0xFFFFF, -1)

    def count_ge(t):
        return jnp.sum(jnp.where(low >= t, 1, 0).astype(jnp.int32))

    def bis(_, carry):
        lo, hi = carry
        mid = (lo + hi) // 2
        ge = count_ge(mid) >= j
        return (jnp.where(ge, mid, lo), jnp.where(ge, hi, mid))

    tlow, _ = lax.fori_loop(0, 20, bis, (jnp.int32(0), jnp.int32(1 << 20)))

    greater = above + count_ge(tlow + 1)
    t_extra = _K - greater
    eqidx = jnp.where(valid & (low == tlow), cidx_ref[...], _POS)

    def count_le(c):
        return jnp.sum(jnp.where(eqidx <= c, 1, 0).astype(jnp.int32))

    def bis_idx(_, carry):
        lo, hi = carry
        mid = (lo + hi) // 2
        ge = count_le(mid) >= t_extra
        return (jnp.where(ge, lo, mid), jnp.where(ge, mid, hi))

    _, cutoff = lax.fori_loop(
        0, 24, bis_idx, (jnp.int32(-1), jnp.int32(_N - 1)))

    u_t = (bstar << 20) | tlow
    t_ref[...] = jnp.full((128,), u_t ^ _MIN32, jnp.int32)
    c_ref[...] = jnp.full((128,), cutoff, jnp.int32)


def _k5_body(t_smem, c_smem, x_ref, o_ref):
    ts = t_smem[0]
    cutoff = c_smem[0]
    x = x_ref[...]
    rows, d = x_ref.shape
    xi = pltpu.bitcast(x, jnp.int32)
    us = jnp.where(xi < 0, xi ^ _POS, xi)
    base = pl.program_id(0) * (rows * d)
    fidx = (base
            + lax.broadcasted_iota(jnp.int32, (rows, d), 0) * d
            + lax.broadcasted_iota(jnp.int32, (rows, d), 1))
    keep = (us > ts) | ((us == ts) & (fidx <= cutoff))
    o_ref[...] = jnp.where(keep, x, jnp.float32(0.0))


def kernel(features):
    b, l, d = features.shape
    n = b * l * d
    flat = features.reshape(n)
    flat_i = lax.bitcast_convert_type(flat, jnp.int32)
    mesh = plsc.VectorSubcoreMesh(
        core_axis_name="c", subcore_axis_name="s", num_cores=2, num_subcores=16
    )

    k1 = functools.partial(
        pl.kernel,
        out_type=jax.ShapeDtypeStruct((_NWORK, _NBKT), jnp.int32),
        mesh=mesh,
        scratch_types=[
            pltpu.VMEM((16384,), jnp.int32),
            pltpu.VMEM((16384,), jnp.int32),
            pltpu.VMEM((16 * _NBKT,), jnp.int32),
            pltpu.VMEM((_NBKT,), jnp.int32),
            pltpu.SemaphoreType.DMA,
            pltpu.SemaphoreType.DMA,
        ],
        compiler_params=pltpu.CompilerParams(needs_layout_passes=False),
    )(_k1_body)
    hist = k1(flat_i)

    b_rep, a_rep = pl.pallas_call(
        _k2_body,
        out_shape=(
            jax.ShapeDtypeStruct((128,), jnp.int32),
            jax.ShapeDtypeStruct((128,), jnp.int32),
        ),
    )(hist)

    k3 = functools.partial(
        pl.kernel,
        out_type=(
            jax.ShapeDtypeStruct((_NWORK, _CAP), jnp.int32),
            jax.ShapeDtypeStruct((_NWORK, _CAP), jnp.int32),
            jax.ShapeDtypeStruct((_NWORK, 128), jnp.int32),
        ),
        mesh=mesh,
        scratch_types=[
            pltpu.VMEM((32768,), jnp.int32),
            pltpu.VMEM((_CAP,), jnp.int32),
            pltpu.VMEM((_CAP,), jnp.int32),
            pltpu.VMEM((16,), jnp.int32),
            pltpu.VMEM((128,), jnp.int32),
        ],
        compiler_params=pltpu.CompilerParams(needs_layout_passes=False),
    )(_k3_body)
    cand, cidx, cnt = k3(flat_i, b_rep)

    tvec, cvec = pl.pallas_call(
        _k4_body,
        in_specs=[
            pl.BlockSpec(memory_space=pltpu.SMEM),
            pl.BlockSpec(memory_space=pltpu.SMEM),
            pl.BlockSpec(memory_space=pltpu.VMEM),
            pl.BlockSpec(memory_space=pltpu.VMEM),
            pl.BlockSpec(memory_space=pltpu.VMEM),
        ],
        out_shape=(
            jax.ShapeDtypeStruct((128,), jnp.int32),
            jax.ShapeDtypeStruct((128,), jnp.int32),
        ),
    )(b_rep, a_rep, cand, cidx, cnt)

    rows = b * l
    blk = 64
    out = pl.pallas_call(
        _k5_body,
        grid=(rows // blk,),
        in_specs=[
            pl.BlockSpec(memory_space=pltpu.SMEM),
            pl.BlockSpec(memory_space=pltpu.SMEM),
            pl.BlockSpec((blk, d), lambda i: (i, 0)),
        ],
        out_specs=pl.BlockSpec((blk, d), lambda i: (i, 0)),
        out_shape=jax.ShapeDtypeStruct((rows, d), jnp.float32),
    )(tvec, cvec, flat.reshape(rows, d))
    return out.reshape(b, l, d)


# speedup vs baseline: 41.7937x; 1.0392x over previous
import functools

import jax
import jax.numpy as jnp
import numpy as np
from jax import lax
from jax.experimental import pallas as pl
from jax.experimental.pallas import tpu as pltpu
from jax.experimental.pallas import tpu_sc as plsc

_K = 16384
_N = 512 * 6 * 4096
_NWORK = 32
_NBKT = 4096
_CAP = 4096
_MIN32 = np.int32(-2147483648)
_POS = np.int32(2147483647)


def _umap(xi):
    return jnp.where(xi < 0, ~xi, xi | _MIN32)


def _k1_body(x_hbm, hist_hbm, buf, buf2, hist1, outv, sem0, sem1):
    n = x_hbm.shape[0]
    shard = n // _NWORK
    win = 16384
    nwin = shard // win
    nvec_u = win // (16 * 8)

    wid = lax.axis_index("s") * 2 + lax.axis_index("c")
    base = wid * shard

    zeros16 = jnp.zeros((16,), jnp.int32)
    ones16 = jnp.ones((16,), jnp.int32)
    laneoff = lax.iota(jnp.int32, 16) * _NBKT

    @plsc.parallel_loop(0, (16 * _NBKT) // 16, unroll=8)
    def _(i):
        hist1[pl.ds(i * 16, 16)] = zeros16

    bufs = (buf, buf2)
    sems = (sem0, sem1)

    def start(w, b):
        pltpu.async_copy(x_hbm.at[pl.ds(base + w * win, win)], bufs[b], sems[b])

    def wait(w, b):
        pltpu.make_async_copy(
            x_hbm.at[pl.ds(base + w * win, win)], bufs[b], sems[b]).wait()

    start(0, 0)

    def win2_body(w2, _):
        for b in range(2):
            w = w2 * 2 + b

            @pl.when(w + 1 < nwin)
            def _():
                start(w + 1, (b + 1) % 2)

            wait(w, b)
            cur = bufs[b]

            @plsc.parallel_loop(0, win // 16, unroll=8)
            def _(i):
                xi = cur[pl.ds(i * 16, 16)]
                m = _umap(xi)
                bkt = lax.shift_right_logical(m, 20)
                plsc.addupdate_scatter(hist1, [laneoff + bkt], ones16)
        return 0

    lax.fori_loop(0, nwin // 2, win2_body, 0)

    @plsc.parallel_loop(0, _NBKT // 16, unroll=2)
    def _(g):
        acc = hist1[pl.ds(g * 16, 16)]
        for r in range(1, 16):
            acc = acc + hist1[pl.ds(r * _NBKT + g * 16, 16)]
        outv[pl.ds(g * 16, 16)] = acc
    pltpu.sync_copy(outv, hist_hbm.at[wid])


def _k2_body(hist_ref, b_ref, a_ref):
    merged = jnp.sum(hist_ref[...], axis=0, keepdims=True)
    bidx = lax.broadcasted_iota(jnp.int32, (1, _NBKT), 1)

    def above(b):
        return jnp.sum(jnp.where(bidx > b, merged, 0))

    def bis(_, carry):
        lo, hi = carry
        mid = (lo + hi) // 2
        less = above(mid) < _K
        return (jnp.where(less, lo, mid), jnp.where(less, mid, hi))

    lo0 = jnp.int32(-1)
    hi0 = jnp.int32(_NBKT - 1)
    _, bstar = lax.fori_loop(0, 12, bis, (lo0, hi0))
    a = above(bstar)
    b_ref[...] = jnp.full((128,), bstar, jnp.int32)
    a_ref[...] = jnp.full((128,), a, jnp.int32)


def _k3_body(x_hbm, b_hbm, cand_hbm, cidx_hbm, cnt_hbm,
             buf, buf2, cand, cidx, bvec, cntv, sem0, sem1):
    n = x_hbm.shape[0]
    shard = n // _NWORK
    win = 32768
    nwin = shard // win
    nvec_u = win // (16 * 4)

    wid = lax.axis_index("s") * 2 + lax.axis_index("c")
    base = wid * shard

    pltpu.sync_copy(b_hbm.at[pl.ds(0, 16)], bvec)
    bv = bvec[...]
    lanes = lax.iota(jnp.int32, 16)

    bufs = (buf, buf2)
    sems = (sem0, sem1)

    def start(w, b):
        pltpu.async_copy(x_hbm.at[pl.ds(base + w * win, win)], bufs[b], sems[b])

    def wait(w, b):
        pltpu.make_async_copy(
            x_hbm.at[pl.ds(base + w * win, win)], bufs[b], sems[b]).wait()

    start(0, 0)

    def win2_body(w2, off):
        for b in range(2):
            w = w2 * 2 + b

            @pl.when(w + 1 < nwin)
            def _():
                start(w + 1, (b + 1) % 2)

            wait(w, b)
            cur = bufs[b]
            wbase = base + w * win

            @plsc.parallel_loop(0, win // 16, unroll=4, carry=off)
            def inner(i, off):
                xi = cur[pl.ds(i * 16, 16)]
                m = _umap(xi)
                bkt = lax.shift_right_logical(m, 20)
                sel = bkt == bv
                offc = jnp.minimum(off, _CAP - 16)
                plsc.store_compressed(cand.at[pl.ds(offc, 16)], m, mask=sel)
                fidx = (wbase + i * 16) + lanes
                plsc.store_compressed(cidx.at[pl.ds(offc, 16)], fidx, mask=sel)
                return off + plsc.all_reduce_population_count(sel)[0]

            off = inner
        return off

    off = lax.fori_loop(0, nwin // 2, win2_body, jnp.int32(0))
    cnt = jnp.minimum(off, _CAP)

    def cnt_body(t, _):
        cntv[pl.ds(t * 16, 16)] = jnp.full((16,), cnt, jnp.int32)
        return 0

    lax.fori_loop(0, 8, cnt_body, 0)
    pltpu.sync_copy(cand, cand_hbm.at[wid])
    pltpu.sync_copy(cidx, cidx_hbm.at[wid])
    pltpu.sync_copy(cntv, cnt_hbm.at[wid])


def _k4_body(b_smem, a_smem, cand_ref, cidx_ref, cnt_ref, t_ref, c_ref):
    bstar = b_smem[0]
    above = a_smem[0]
    j = _K - above

    cnt0 = cnt_ref[...][:, 0:1]
    valid = lax.broadcasted_iota(jnp.int32, (_NWORK, _CAP), 1) < cnt0
    low = jnp.where(valid, cand_ref[...] & 0xFFFFF, -1)

    def count_ge(t):
        return jnp.sum(jnp.where(low >= t, 1, 0).astype(jnp.int32))

    def bis(_, carry):
        lo, hi = carry
        mid = (lo + hi) // 2
        ge = count_ge(mid) >= j
        return (jnp.where(ge, mid, lo), jnp.where(ge, hi, mid))

    tlow, _ = lax.fori_loop(0, 20, bis, (jnp.int32(0), jnp.int32(1 << 20)))

    greater = above + count_ge(tlow + 1)
    t_extra = _K - greater
    eqidx = jnp.where(valid & (low == tlow), cidx_ref[...], _POS)

    def count_le(c):
        return jnp.sum(jnp.where(eqidx <= c, 1, 0).astype(jnp.int32))

    def bis_idx(_, carry):
        lo, hi = carry
        mid = (lo + hi) // 2
        ge = count_le(mid) >= t_extra
        return (jnp.where(ge, lo, mid), jnp.where(ge, mid, hi))

    _, cutoff = lax.fori_loop(
        0, 24, bis_idx, (jnp.int32(-1), jnp.int32(_N - 1)))

    u_t = (bstar << 20) | tlow
    t_ref[...] = jnp.full((128,), u_t ^ _MIN32, jnp.int32)
    c_ref[...] = jnp.full((128,), cutoff, jnp.int32)


def _k5_body(t_smem, c_smem, x_ref, o_ref):
    ts = t_smem[0]
    cutoff = c_smem[0]
    x = x_ref[...]
    rows, d = x_ref.shape
    xi = pltpu.bitcast(x, jnp.int32)
    us = jnp.where(xi < 0, xi ^ _POS, xi)
    base = pl.program_id(0) * (rows * d)
    fidx = (base
            + lax.broadcasted_iota(jnp.int32, (rows, d), 0) * d
            + lax.broadcasted_iota(jnp.int32, (rows, d), 1))
    keep = (us > ts) | ((us == ts) & (fidx <= cutoff))
    o_ref[...] = jnp.where(keep, x, jnp.float32(0.0))


def kernel(features):
    b, l, d = features.shape
    n = b * l * d
    flat = features.reshape(n)
    flat_i = lax.optimization_barrier(lax.bitcast_convert_type(flat, jnp.int32))
    mesh = plsc.VectorSubcoreMesh(
        core_axis_name="c", subcore_axis_name="s", num_cores=2, num_subcores=16
    )

    k1 = functools.partial(
        pl.kernel,
        out_type=jax.ShapeDtypeStruct((_NWORK, _NBKT), jnp.int32),
        mesh=mesh,
        scratch_types=[
            pltpu.VMEM((16384,), jnp.int32),
            pltpu.VMEM((16384,), jnp.int32),
            pltpu.VMEM((16 * _NBKT,), jnp.int32),
            pltpu.VMEM((_NBKT,), jnp.int32),
            pltpu.SemaphoreType.DMA,
            pltpu.SemaphoreType.DMA,
        ],
        compiler_params=pltpu.CompilerParams(needs_layout_passes=False),
    )(_k1_body)
    hist = k1(flat_i)

    b_rep, a_rep = pl.pallas_call(
        _k2_body,
        out_shape=(
            jax.ShapeDtypeStruct((128,), jnp.int32),
            jax.ShapeDtypeStruct((128,), jnp.int32),
        ),
    )(hist)

    k3 = functools.partial(
        pl.kernel,
        out_type=(
            jax.ShapeDtypeStruct((_NWORK, _CAP), jnp.int32),
            jax.ShapeDtypeStruct((_NWORK, _CAP), jnp.int32),
            jax.ShapeDtypeStruct((_NWORK, 128), jnp.int32),
        ),
        mesh=mesh,
        scratch_types=[
            pltpu.VMEM((32768,), jnp.int32),
            pltpu.VMEM((_CAP,), jnp.int32),
            pltpu.VMEM((_CAP,), jnp.int32),
            pltpu.VMEM((16,), jnp.int32),
            pltpu.VMEM((128,), jnp.int32),
        ],
        compiler_params=pltpu.CompilerParams(needs_layout_passes=False),
    )(_k3_body)
    cand, cidx, cnt = k3(flat_i, b_rep)

    tvec, cvec = pl.pallas_call(
        _k4_body,
        in_specs=[
            pl.BlockSpec(memory_space=pltpu.SMEM),
            pl.BlockSpec(memory_space=pltpu.SMEM),
            pl.BlockSpec(memory_space=pltpu.VMEM),
            pl.BlockSpec(memory_space=pltpu.VMEM),
            pl.BlockSpec(memory_space=pltpu.VMEM),
        ],
        out_shape=(
            jax.ShapeDtypeStruct((128,), jnp.int32),
            jax.ShapeDtypeStruct((128,), jnp.int32),
        ),
    )(b_rep, a_rep, cand, cidx, cnt)

    rows = b * l
    blk = 256
    out = pl.pallas_call(
        _k5_body,
        grid=(rows // blk,),
        in_specs=[
            pl.BlockSpec(memory_space=pltpu.SMEM),
            pl.BlockSpec(memory_space=pltpu.SMEM),
            pl.BlockSpec((blk, d), lambda i: (i, 0)),
        ],
        out_specs=pl.BlockSpec((blk, d), lambda i: (i, 0)),
        out_shape=jax.ShapeDtypeStruct((rows, d), jnp.float32),
    )(tvec, cvec, flat.reshape(rows, d))
    return out.reshape(b, l, d)
